# Initial kernel scaffold; baseline (speedup 1.0000x reference)
import jax
import jax.numpy as jnp
from jax.experimental import pallas as pl


def kernel(x, edge_weight, enc1_W, enc1_b, enc1_g, enc1_beta, enc2_W, enc2_b, enc2_g, enc2_beta, gcf_W, gcf_b, gcm_W, gcm_b, gcv_W, gcv_b, dec_W, dec_b, dec_g, dec_beta, edge_index):
    raise NotImplementedError("write your pallas kernel here")


# speedup vs baseline: 1.3372x; 1.3372x over previous
import functools

import jax
import jax.numpy as jnp
import numpy as np
from jax.experimental import pallas as pl
from jax.experimental.pallas import tpu as pltpu

N_NODES = 10000
N_EDGES = 320000
D_IN = 128
BN_EPS = 0.001
MAX_NEG = N_EDGES + N_NODES

_BN_S = 1.0 / np.sqrt(1.0 + BN_EPS)


def _elu(h):
    return jnp.where(h > 0, h, jnp.exp(jnp.minimum(h, 0.0)) - 1.0)


def _encoder_body(x_ref, w1_ref, b1_ref, g1_ref, be1_ref, w2_ref, b2_ref,
                  g2_ref, be2_ref, wf_ref, feat_ref, xwf_ref):
    x = x_ref[...]
    s1 = g1_ref[...] * _BN_S
    h = jax.lax.dot_general(x, w1_ref[...], (((1,), (1,)), ((), ())),
                            preferred_element_type=jnp.float32)
    h = _elu(h * s1[None, :] + (b1_ref[...] * s1 + be1_ref[...])[None, :])
    s2 = g2_ref[...] * _BN_S
    f = jax.lax.dot_general(h, w2_ref[...], (((1,), (1,)), ((), ())),
                            preferred_element_type=jnp.float32)
    f = _elu(f * s2[None, :] + (b2_ref[...] * s2 + be2_ref[...])[None, :])
    feat_ref[...] = f
    xwf_ref[...] = jax.lax.dot_general(f, wf_ref[...], (((1,), (1,)), ((), ())),
                                       preferred_element_type=jnp.float32)


def _encoder(x, enc1_W, enc1_b, enc1_g, enc1_beta, enc2_W, enc2_b, enc2_g,
             enc2_beta, gcf_W):
    blk = 2000
    grid = N_NODES // blk
    full = lambda s: pl.BlockSpec(s, lambda i: (0,) * len(s))
    return pl.pallas_call(
        _encoder_body,
        grid=(grid,),
        in_specs=[
            pl.BlockSpec((blk, D_IN), lambda i: (i, 0)),
            full((256, 128)), full((256,)), full((256,)), full((256,)),
            full((64, 256)), full((64,)), full((64,)), full((64,)),
            full((32, 64)),
        ],
        out_specs=[
            pl.BlockSpec((blk, 64), lambda i: (i, 0)),
            pl.BlockSpec((blk, 32), lambda i: (i, 0)),
        ],
        out_shape=[
            jax.ShapeDtypeStruct((N_NODES, 64), jnp.float32),
            jax.ShapeDtypeStruct((N_NODES, 32), jnp.float32),
        ],
    )(x, enc1_W, enc1_b, enc1_g, enc1_beta, enc2_W, enc2_b, enc2_g, enc2_beta,
      gcf_W)


def kernel(x, edge_weight, enc1_W, enc1_b, enc1_g, enc1_beta, enc2_W, enc2_b,
           enc2_g, enc2_beta, gcf_W, gcf_b, gcm_W, gcm_b, gcv_W, gcv_b, dec_W,
           dec_b, dec_g, dec_beta, edge_index):
    n = N_NODES
    src, dst = edge_index[0], edge_index[1]

    feat_x, xw_f = _encoder(x, enc1_W, enc1_b, enc1_g, enc1_beta, enc2_W,
                            enc2_b, enc2_g, enc2_beta, gcf_W)

    deg = jnp.ones((n,), jnp.float32).at[dst].add(edge_weight)
    dinv = 1.0 / jnp.sqrt(deg)
    norm = dinv[src] * edge_weight * dinv[dst]

    agg1 = jnp.zeros((n, 32), jnp.float32).at[dst].add(xw_f[src] * norm[:, None])
    h = jax.nn.relu(agg1 + dinv[:, None] ** 2 * xw_f + gcf_b[None, :])

    hw = jnp.concatenate([h @ gcm_W.T, h @ gcv_W.T], axis=1)
    agg2 = jnp.zeros((n, 32), jnp.float32).at[dst].add(hw[src] * norm[:, None])
    out2 = agg2 + dinv[:, None] ** 2 * hw
    mu = out2[:, :16] + gcm_b[None, :]
    logstd = jnp.minimum(out2[:, 16:] + gcv_b[None, :], 10.0)

    feat = jnp.concatenate([feat_x, mu], axis=1)

    sd = dec_g * _BN_S
    xd = feat @ dec_W.T
    xd = _elu(xd * sd[None, :] + (dec_b * sd + dec_beta)[None, :])
    dae_loss = jnp.mean((xd - x) ** 2)

    p = jax.nn.sigmoid(jnp.sum(feat[src] * feat[dst], axis=1))
    pos_loss = jnp.mean(p - p * edge_weight + jnp.log1p(jnp.exp(-p)))

    nk1, nk2 = jax.random.split(jax.random.key(12345))
    neg_src = jax.random.randint(nk1, (MAX_NEG,), 0, n, dtype=jnp.int32)
    neg_dst = jax.random.randint(nk2, (MAX_NEG,), 0, n, dtype=jnp.int32)
    t = jax.nn.sigmoid(jnp.sum(feat[neg_src] * feat[neg_dst], axis=1))
    num_neg = jnp.sum(src != dst) + n
    mask = jnp.arange(MAX_NEG) < num_neg
    neg_loss = (jnp.sum(jnp.where(mask, jnp.log1p(jnp.exp(t)), 0.0))
                / num_neg.astype(jnp.float32))

    kl = -0.5 * jnp.mean(jnp.sum(1.0 + 2.0 * logstd - mu ** 2
                                 - jnp.exp(2.0 * logstd), axis=1))
    gae_loss = pos_loss + neg_loss + (1.0 / n) * kl
    return feat, dae_loss, gae_loss
